# baseline (speedup 1.0000x reference)
import functools
import math

import jax
import jax.numpy as jnp
from jax import lax
from jax.experimental import pallas as pl
from jax.experimental.pallas import tpu as pltpu
from jax.experimental.pallas import tpu_sc as plsc

N = 10000
E = 320000
D = 128
DE = 16
H = 128
G = 8
GF = 4
OUT = 8
L = 3
EPS = 1e-7
BN_EPS = 1e-5
BN_SCALE = 1.0 / math.sqrt(1.0 + BN_EPS)

NC = 2
NS = 16
NW = NC * NS
EPT = E // NW
C = 80
NCH = EPT // C
RPT = 624
TAIL = N - NS * RPT


def _sc_mesh():
    return plsc.VectorSubcoreMesh(
        core_axis_name="c", subcore_axis_name="s",
        num_cores=NC, num_subcores=NS)


def _cnt_body(dst_hbm, out_hbm, accum, dst_flat, ids, ones_v, sem_s):
    c = lax.axis_index("c")
    s = lax.axis_index("s")
    wid = s * NC + c

    cp_d = pltpu.async_copy(dst_hbm.at[pl.ds(wid * EPT, EPT)], dst_flat,
                            sem_s.at[0])

    def _fill(r, carry):
        for k in range(8):
            ones_v[r, pl.ds(k * 16, 16)] = jnp.zeros((16,), jnp.float32)
        return carry

    lax.fori_loop(0, C, _fill, 0)
    cp_d.wait()
    NZC = RPT // C
    remc = RPT - NZC * C
    for j in range(NZC):
        pltpu.async_copy(ones_v, accum.at[pl.ds(s * RPT + j * C, C)],
                         sem_s.at[j % 5])
    pltpu.async_copy(ones_v.at[pl.ds(0, remc)],
                     accum.at[pl.ds(s * RPT + NZC * C, remc)],
                     sem_s.at[NZC % 5])

    @pl.when(s == NS - 1)
    def _zero_tail():
        pltpu.async_copy(ones_v.at[pl.ds(0, TAIL)],
                         accum.at[pl.ds(NS * RPT, TAIL)],
                         sem_s.at[(NZC + 1) % 5])

    for j in range(NZC):
        pltpu.make_async_copy(out_hbm.at[0, pl.ds(0, C)], ones_v,
                              sem_s.at[j % 5]).wait()
    pltpu.make_async_copy(out_hbm.at[0, pl.ds(0, remc)],
                          ones_v.at[pl.ds(0, remc)], sem_s.at[NZC % 5]).wait()

    @pl.when(s == NS - 1)
    def _zero_tail_wait():
        pltpu.make_async_copy(out_hbm.at[0, pl.ds(0, TAIL)],
                              ones_v.at[pl.ds(0, TAIL)],
                              sem_s.at[(NZC + 1) % 5]).wait()

    def _fill1(r, carry):
        for k in range(8):
            ones_v[r, pl.ds(k * 16, 16)] = jnp.ones((16,), jnp.float32)
        return carry

    lax.fori_loop(0, C, _fill1, 0)
    plsc.subcore_barrier()

    CSLOT = 5

    def _chunk(J, carry):
        for j in range(CSLOT):
            g = J * CSLOT + j

            @pl.when(g >= CSLOT)
            def _drain():
                pltpu.make_async_copy(out_hbm.at[0, pl.ds(0, C)],
                                      ones_v, sem_s.at[j]).wait()

            for k in range(C // 16):
                ids[j][pl.ds(k * 16, 16)] = dst_flat[pl.ds(g * C + k * 16, 16)]
            pltpu.async_copy(ones_v, accum.at[ids[j]], sem_s.at[j], add=True)
        return carry

    lax.fori_loop(0, NCH // CSLOT, _chunk, 0)
    for j in range(CSLOT):
        pltpu.make_async_copy(out_hbm.at[0, pl.ds(0, C)], ones_v,
                              sem_s.at[j]).wait()

    plsc.subcore_barrier()
    pltpu.sync_copy(accum.at[pl.ds(s * RPT, RPT)],
                    out_hbm.at[c, pl.ds(s * RPT, RPT)])

    @pl.when(s == NS - 1)
    def _dump_tail():
        pltpu.sync_copy(accum.at[pl.ds(NS * RPT, TAIL)],
                        out_hbm.at[c, pl.ds(NS * RPT, TAIL)])


def _cnt_entry(dst_hbm, out_hbm, accum, dst_flat, id0, id1, id2, id3, id4,
               ones_v, sem_s):
    _cnt_body(dst_hbm, out_hbm, accum, dst_flat, [id0, id1, id2, id3, id4],
              ones_v, sem_s)


@functools.cache
def _cnt_kernel():
    return pl.kernel(
        _cnt_entry,
        out_type=jax.ShapeDtypeStruct((NC, N, D), jnp.float32),
        mesh=_sc_mesh(),
        scratch_types=[
            pltpu.VMEM_SHARED((N, D), jnp.float32),
            pltpu.VMEM((EPT,), jnp.int32),
            pltpu.VMEM((C,), jnp.int32),
            pltpu.VMEM((C,), jnp.int32),
            pltpu.VMEM((C,), jnp.int32),
            pltpu.VMEM((C,), jnp.int32),
            pltpu.VMEM((C,), jnp.int32),
            pltpu.VMEM((C, D), jnp.float32),
            pltpu.SemaphoreType.DMA((5,)),
        ],
    )


MC = 16
MNCH = EPT // MC
NSLOT = 5
MJ = MNCH // NSLOT


def _msgpass_body(layer, h_hbm, src_hbm, dst_hbm, e_hbm, out_hbm,
                  accum, src_flat, dst_flat, idss, idds, rvs, evs,
                  sem_g, sem_e, sem_s):
    c = lax.axis_index("c")
    s = lax.axis_index("s")
    wid = s * NC + c

    cp_s = pltpu.async_copy(src_hbm.at[pl.ds(wid * EPT, EPT)], src_flat,
                            sem_g.at[0])
    cp_d = pltpu.async_copy(dst_hbm.at[pl.ds(wid * EPT, EPT)], dst_flat,
                            sem_e.at[0])

    zb = rvs[0]

    def _fill_zb(r, carry):
        for k in range(8):
            zb[r, pl.ds(k * 16, 16)] = jnp.zeros((16,), jnp.float32)
        return carry

    lax.fori_loop(0, MC, _fill_zb, 0)
    NZ = RPT // MC
    for j in range(NZ):
        pltpu.async_copy(zb, accum.at[pl.ds(s * RPT + j * MC, MC)],
                         sem_s.at[j % NSLOT])

    @pl.when(s == NS - 1)
    def _zero_tail():
        pltpu.async_copy(zb.at[pl.ds(0, TAIL)],
                         accum.at[pl.ds(NS * RPT, TAIL)],
                         sem_s.at[NZ % NSLOT])

    cp_s.wait()
    cp_d.wait()
    for j in range(NZ):
        pltpu.make_async_copy(h_hbm.at[pl.ds(0, MC)], rvs[0],
                              sem_s.at[j % NSLOT]).wait()

    @pl.when(s == NS - 1)
    def _zero_tail_wait():
        pltpu.make_async_copy(h_hbm.at[pl.ds(0, TAIL)],
                              rvs[0].at[pl.ds(0, TAIL)],
                              sem_s.at[NZ % NSLOT]).wait()

    plsc.subcore_barrier()

    def _issue(g, slot):
        idss[slot][pl.ds(0, MC)] = src_flat[pl.ds(g * MC, MC)]
        pltpu.async_copy(h_hbm.at[idss[slot]], rvs[slot], sem_g.at[slot])
        pltpu.async_copy(e_hbm.at[layer, pl.ds(wid * (EPT // 2) + g * (MC // 2),
                                               MC // 2)],
                         evs[slot], sem_e.at[slot])

    _HI = jnp.full((16,), 0xFFFF0000, dtype=jnp.uint32)

    def _finish(g, slot):
        pltpu.make_async_copy(h_hbm.at[pl.ds(0, MC)], rvs[slot],
                              sem_g.at[slot]).wait()
        pltpu.make_async_copy(e_hbm.at[layer, pl.ds(0, MC // 2)], evs[slot],
                              sem_e.at[slot]).wait()
        rv = rvs[slot]
        ev = evs[slot]

        def _pair(p):
            r0 = 2 * p
            for k in range(8):
                sl = pl.ds(k * 16, 16)
                w = ev[p, sl]
                ea = jax.lax.bitcast_convert_type(w << 16, jnp.float32)
                eb = jax.lax.bitcast_convert_type(w & _HI, jnp.float32)
                rv[r0, sl] = jnp.maximum(rv[r0, sl] + ea, 0.0)
                rv[r0 + 1, sl] = jnp.maximum(rv[r0 + 1, sl] + eb, 0.0)

        plsc.parallel_loop(0, MC // 2, unroll=4)(_pair)
        idds[slot][pl.ds(0, MC)] = dst_flat[pl.ds(g * MC, MC)]
        pltpu.async_copy(rv, accum.at[idds[slot]], sem_s.at[slot], add=True)

    LOOKAHEAD = 4

    def _outer(J, carry):
        for j in range(NSLOT):
            g = J * NSLOT + j
            slot = j
            pslot = (j - LOOKAHEAD) % NSLOT

            @pl.when(g >= NSLOT)
            def _drain_scatter():
                pltpu.make_async_copy(h_hbm.at[pl.ds(0, MC)],
                                      rvs[slot], sem_s.at[slot]).wait()

            _issue(g, slot)

            @pl.when(g >= LOOKAHEAD)
            def _do_finish():
                _finish(g - LOOKAHEAD, pslot)
        return carry

    lax.fori_loop(0, MJ, _outer, 0)
    for t in range(LOOKAHEAD, 0, -1):
        _finish(MNCH - t, (MNCH - t) % NSLOT)
    for j in range(NSLOT):
        pltpu.make_async_copy(h_hbm.at[pl.ds(0, MC)], rvs[j],
                              sem_s.at[j]).wait()

    plsc.subcore_barrier()
    pltpu.sync_copy(accum.at[pl.ds(s * RPT, RPT)],
                    out_hbm.at[c, pl.ds(s * RPT, RPT)])

    @pl.when(s == NS - 1)
    def _dump_tail():
        pltpu.sync_copy(accum.at[pl.ds(NS * RPT, TAIL)],
                        out_hbm.at[c, pl.ds(NS * RPT, TAIL)])


def _msgpass_entry(layer):
    def body(h_hbm, src_hbm, dst_hbm, e_hbm, out_hbm, accum,
             src_flat, dst_flat,
             is0, is1, is2, is3, is4, id0, id1, id2, id3, id4,
             rv0, rv1, rv2, rv3, rv4, ev0, ev1, ev2, ev3, ev4,
             sem_g, sem_e, sem_s):
        _msgpass_body(layer, h_hbm, src_hbm, dst_hbm, e_hbm, out_hbm,
                      accum, src_flat, dst_flat,
                      [is0, is1, is2, is3, is4], [id0, id1, id2, id3, id4],
                      [rv0, rv1, rv2, rv3, rv4], [ev0, ev1, ev2, ev3, ev4],
                      sem_g, sem_e, sem_s)
    return body


@functools.cache
def _msgpass_call(layer):
    return pl.kernel(
        _msgpass_entry(layer),
        out_type=jax.ShapeDtypeStruct((NC, N, D), jnp.float32),
        mesh=_sc_mesh(),
        scratch_types=(
            [pltpu.VMEM_SHARED((N, D), jnp.float32),
             pltpu.VMEM((EPT,), jnp.int32),
             pltpu.VMEM((EPT,), jnp.int32)]
            + [pltpu.VMEM((MC,), jnp.int32) for _ in range(2 * NSLOT)]
            + [pltpu.VMEM((MC, D), jnp.float32) for _ in range(NSLOT)]
            + [pltpu.VMEM((MC // 2, D), jnp.uint32) for _ in range(NSLOT)]
            + [pltpu.SemaphoreType.DMA((NSLOT,)),
               pltpu.SemaphoreType.DMA((NSLOT,)),
               pltpu.SemaphoreType.DMA((NSLOT,))]
        ),
    )


_BE = 2000


def _edge_emb_body(ea, w, b, o):
    ee = jnp.dot(ea[:, :DE], w[0], preferred_element_type=jnp.float32) + b[0]
    eo = jnp.dot(ea[:, DE:], w[0], preferred_element_type=jnp.float32) + b[0]
    ue = jax.lax.bitcast_convert_type(
        ee.astype(jnp.bfloat16), jnp.uint16).astype(jnp.uint32)
    uo = jax.lax.bitcast_convert_type(
        eo.astype(jnp.bfloat16), jnp.uint16).astype(jnp.uint32)
    o[0] = ue | (uo << 16)


def _edge_emb(edge_attr2, We, be):
    return pl.pallas_call(
        _edge_emb_body,
        grid=(L, E // 2 // _BE),
        in_specs=[
            pl.BlockSpec((_BE, 2 * DE), lambda l, e: (e, 0)),
            pl.BlockSpec((1, DE, H), lambda l, e: (l, 0, 0)),
            pl.BlockSpec((1, 1, H), lambda l, e: (l, 0, 0)),
        ],
        out_specs=pl.BlockSpec((1, _BE, H), lambda l, e: (l, e, 0)),
        out_shape=jax.ShapeDtypeStruct((L, E // 2, H), jnp.uint32),
    )(edge_attr2, We, be)


_BN = 1000


def _layer_body(sp, cp, h, w1, b1, g1, c1, w2, b2, o):
    cnt = cp[0, :, 0:1] + cp[1, :, 0:1]
    ssum = sp[0] + sp[1] + EPS * cnt
    agg = ssum / jnp.maximum(cnt, 1.0)
    out = agg + h[...]
    t = jnp.dot(out, w1[...], preferred_element_type=jnp.float32) + b1[...]
    t = t * (g1[...] * BN_SCALE) + c1[...]
    t = jnp.maximum(t, 0.0)
    hn = jnp.dot(t, w2[...], preferred_element_type=jnp.float32) + b2[...]
    o[...] = jnp.maximum(hn, 0.0)


def _layer_update(sparts, cparts, h, w1, b1, g1, c1, w2, b2):
    return pl.pallas_call(
        _layer_body,
        grid=(N // _BN,),
        in_specs=[
            pl.BlockSpec((NC, _BN, D), lambda b: (0, b, 0)),
            pl.BlockSpec((NC, _BN, D), lambda b: (0, b, 0)),
            pl.BlockSpec((_BN, D), lambda b: (b, 0)),
            pl.BlockSpec((D, 2 * H), lambda b: (0, 0)),
            pl.BlockSpec((1, 2 * H), lambda b: (0, 0)),
            pl.BlockSpec((1, 2 * H), lambda b: (0, 0)),
            pl.BlockSpec((1, 2 * H), lambda b: (0, 0)),
            pl.BlockSpec((2 * H, H), lambda b: (0, 0)),
            pl.BlockSpec((1, H), lambda b: (0, 0)),
        ],
        out_specs=pl.BlockSpec((_BN, D), lambda b: (b, 0)),
        out_shape=jax.ShapeDtypeStruct((N, D), jnp.float32),
    )(sparts, cparts, h, w1, b1, g1, c1, w2, b2)


def _head_body(h, b, gf, wh, bhh, o):
    hv = h[...]
    bi = b[...]
    rows = []
    for g in range(G):
        m = (bi == g).astype(jnp.float32)
        cnt = jnp.sum(m, axis=0, keepdims=True)
        sm = jnp.sum(hv * m, axis=0, keepdims=True)
        rows.append(jnp.where(cnt > 0.0, sm / jnp.maximum(cnt, 1.0), 0.0))
    pooled = jnp.concatenate(rows, axis=0)
    z = jnp.concatenate([pooled, gf[...]], axis=1)
    logits = jnp.dot(z, wh[...], preferred_element_type=jnp.float32) + bhh[...]
    mx = jnp.max(logits, axis=1, keepdims=True)
    ex = jnp.exp(logits - mx)
    o[...] = ex / jnp.sum(ex, axis=1, keepdims=True)


def _head(h, batch2d, gf, Wh, bh2d):
    return pl.pallas_call(
        _head_body,
        out_shape=jax.ShapeDtypeStruct((G, OUT), jnp.float32),
    )(h, batch2d, gf, Wh, bh2d)


def kernel(x, edge_attr, graph_features, We, be, Wm1, bm1, bnw, bnb,
           Wm2, bm2, Wh, bh, edge_index, batch, num_graphs):
    src = edge_index[0]
    dst = edge_index[1]
    e_all = _edge_emb(edge_attr.reshape(E // 2, 2 * DE), We,
                      be.reshape(L, 1, H))
    cparts = _cnt_kernel()(dst)
    h = x
    for i in range(L):
        sparts = _msgpass_call(i)(h, src, dst, e_all)
        h = _layer_update(
            sparts, cparts, h,
            Wm1[i], bm1[i].reshape(1, -1), bnw[i].reshape(1, -1),
            bnb[i].reshape(1, -1), Wm2[i], bm2[i].reshape(1, -1))
    return _head(h, batch.reshape(-1, 1), graph_features, Wh,
                 bh.reshape(1, -1))


# speedup vs baseline: 1.1197x; 1.1197x over previous
import functools
import math

import jax
import jax.numpy as jnp
from jax import lax
from jax.experimental import pallas as pl
from jax.experimental.pallas import tpu as pltpu
from jax.experimental.pallas import tpu_sc as plsc

N = 10000
E = 320000
D = 128
DE = 16
H = 128
G = 8
GF = 4
OUT = 8
L = 3
EPS = 1e-7
BN_EPS = 1e-5
BN_SCALE = 1.0 / math.sqrt(1.0 + BN_EPS)

NC = 2
NS = 16
NW = NC * NS
EPT = E // NW
C = 80
NCH = EPT // C
RPT = 624
TAIL = N - NS * RPT


def _sc_mesh():
    return plsc.VectorSubcoreMesh(
        core_axis_name="c", subcore_axis_name="s",
        num_cores=NC, num_subcores=NS)


def _cnt_body(dst_hbm, out_hbm, accum, dst_flat, ids, ones_v, sem_s):
    c = lax.axis_index("c")
    s = lax.axis_index("s")
    wid = s * NC + c

    cp_d = pltpu.async_copy(dst_hbm.at[pl.ds(wid * EPT, EPT)], dst_flat,
                            sem_s.at[0])

    def _fill(r, carry):
        for k in range(8):
            ones_v[r, pl.ds(k * 16, 16)] = jnp.zeros((16,), jnp.float32)
        return carry

    lax.fori_loop(0, C, _fill, 0)
    cp_d.wait()
    NZC = RPT // C
    remc = RPT - NZC * C
    for j in range(NZC):
        pltpu.async_copy(ones_v, accum.at[pl.ds(s * RPT + j * C, C)],
                         sem_s.at[j % 5])
    pltpu.async_copy(ones_v.at[pl.ds(0, remc)],
                     accum.at[pl.ds(s * RPT + NZC * C, remc)],
                     sem_s.at[NZC % 5])

    @pl.when(s == NS - 1)
    def _zero_tail():
        pltpu.async_copy(ones_v.at[pl.ds(0, TAIL)],
                         accum.at[pl.ds(NS * RPT, TAIL)],
                         sem_s.at[(NZC + 1) % 5])

    for j in range(NZC):
        pltpu.make_async_copy(out_hbm.at[0, pl.ds(0, C)], ones_v,
                              sem_s.at[j % 5]).wait()
    pltpu.make_async_copy(out_hbm.at[0, pl.ds(0, remc)],
                          ones_v.at[pl.ds(0, remc)], sem_s.at[NZC % 5]).wait()

    @pl.when(s == NS - 1)
    def _zero_tail_wait():
        pltpu.make_async_copy(out_hbm.at[0, pl.ds(0, TAIL)],
                              ones_v.at[pl.ds(0, TAIL)],
                              sem_s.at[(NZC + 1) % 5]).wait()

    def _fill1(r, carry):
        for k in range(8):
            ones_v[r, pl.ds(k * 16, 16)] = jnp.ones((16,), jnp.float32)
        return carry

    lax.fori_loop(0, C, _fill1, 0)
    plsc.subcore_barrier()

    CSLOT = 5

    def _chunk(J, carry):
        for j in range(CSLOT):
            g = J * CSLOT + j

            @pl.when(g >= CSLOT)
            def _drain():
                pltpu.make_async_copy(out_hbm.at[0, pl.ds(0, C)],
                                      ones_v, sem_s.at[j]).wait()

            for k in range(C // 16):
                ids[j][pl.ds(k * 16, 16)] = dst_flat[pl.ds(g * C + k * 16, 16)]
            pltpu.async_copy(ones_v, accum.at[ids[j]], sem_s.at[j], add=True)
        return carry

    lax.fori_loop(0, NCH // CSLOT, _chunk, 0)
    for j in range(CSLOT):
        pltpu.make_async_copy(out_hbm.at[0, pl.ds(0, C)], ones_v,
                              sem_s.at[j]).wait()

    plsc.subcore_barrier()
    pltpu.sync_copy(accum.at[pl.ds(s * RPT, RPT)],
                    out_hbm.at[c, pl.ds(s * RPT, RPT)])

    @pl.when(s == NS - 1)
    def _dump_tail():
        pltpu.sync_copy(accum.at[pl.ds(NS * RPT, TAIL)],
                        out_hbm.at[c, pl.ds(NS * RPT, TAIL)])


def _cnt_entry(dst_hbm, out_hbm, accum, dst_flat, id0, id1, id2, id3, id4,
               ones_v, sem_s):
    _cnt_body(dst_hbm, out_hbm, accum, dst_flat, [id0, id1, id2, id3, id4],
              ones_v, sem_s)


@functools.cache
def _cnt_kernel():
    return pl.kernel(
        _cnt_entry,
        out_type=jax.ShapeDtypeStruct((NC, N, D), jnp.float32),
        mesh=_sc_mesh(),
        scratch_types=[
            pltpu.VMEM_SHARED((N, D), jnp.float32),
            pltpu.VMEM((EPT,), jnp.int32),
            pltpu.VMEM((C,), jnp.int32),
            pltpu.VMEM((C,), jnp.int32),
            pltpu.VMEM((C,), jnp.int32),
            pltpu.VMEM((C,), jnp.int32),
            pltpu.VMEM((C,), jnp.int32),
            pltpu.VMEM((C, D), jnp.float32),
            pltpu.SemaphoreType.DMA((5,)),
        ],
    )


MC = 16
MNCH = EPT // MC
NSLOT = 5
MJ = MNCH // NSLOT


def _msgpass_body(layer, h_hbm, src_hbm, dst_hbm, e_hbm, out_hbm,
                  accum, src_flat, dst_flat, idss, idds, rvs, evs,
                  sem_g, sem_e, sem_s):
    c = lax.axis_index("c")
    s = lax.axis_index("s")
    wid = s * NC + c

    cp_s = pltpu.async_copy(src_hbm.at[pl.ds(wid * EPT, EPT)], src_flat,
                            sem_g.at[0])
    cp_d = pltpu.async_copy(dst_hbm.at[pl.ds(wid * EPT, EPT)], dst_flat,
                            sem_e.at[0])

    zb = rvs[0]

    def _fill_zb(r, carry):
        for k in range(8):
            zb[r, pl.ds(k * 16, 16)] = jnp.zeros((16,), jnp.float32)
        return carry

    lax.fori_loop(0, MC, _fill_zb, 0)
    NZ = RPT // MC
    for j in range(NZ):
        pltpu.async_copy(zb, accum.at[pl.ds(s * RPT + j * MC, MC)],
                         sem_s.at[j % NSLOT])

    @pl.when(s == NS - 1)
    def _zero_tail():
        pltpu.async_copy(zb.at[pl.ds(0, TAIL)],
                         accum.at[pl.ds(NS * RPT, TAIL)],
                         sem_s.at[NZ % NSLOT])

    cp_s.wait()
    cp_d.wait()
    for j in range(NZ):
        pltpu.make_async_copy(h_hbm.at[pl.ds(0, MC)], rvs[0],
                              sem_s.at[j % NSLOT]).wait()

    @pl.when(s == NS - 1)
    def _zero_tail_wait():
        pltpu.make_async_copy(h_hbm.at[pl.ds(0, TAIL)],
                              rvs[0].at[pl.ds(0, TAIL)],
                              sem_s.at[NZ % NSLOT]).wait()

    plsc.subcore_barrier()

    def _issue(g, slot):
        idss[slot][pl.ds(0, MC)] = src_flat[pl.ds(g * MC, MC)]
        pltpu.async_copy(h_hbm.at[idss[slot]], rvs[slot], sem_g.at[slot])
        pltpu.async_copy(e_hbm.at[layer, pl.ds(wid * (EPT // 2) + g * (MC // 2),
                                               MC // 2)],
                         evs[slot], sem_e.at[slot])

    _HI = jnp.full((16,), 0xFFFF0000, dtype=jnp.uint32)

    def _finish(g, slot):
        pltpu.make_async_copy(h_hbm.at[pl.ds(0, MC)], rvs[slot],
                              sem_g.at[slot]).wait()
        pltpu.make_async_copy(e_hbm.at[layer, pl.ds(0, MC // 2)], evs[slot],
                              sem_e.at[slot]).wait()
        rv = rvs[slot]
        ev = evs[slot]

        def _pair(p):
            r0 = 2 * p
            for k in range(8):
                sl = pl.ds(k * 16, 16)
                w = ev[p, sl]
                ea = jax.lax.bitcast_convert_type(w << 16, jnp.float32)
                eb = jax.lax.bitcast_convert_type(w & _HI, jnp.float32)
                rv[r0, sl] = jnp.maximum(rv[r0, sl] + ea, 0.0)
                rv[r0 + 1, sl] = jnp.maximum(rv[r0 + 1, sl] + eb, 0.0)

        plsc.parallel_loop(0, MC // 2, unroll=2)(_pair)
        idds[slot][pl.ds(0, MC)] = dst_flat[pl.ds(g * MC, MC)]
        pltpu.async_copy(rv, accum.at[idds[slot]], sem_s.at[slot], add=True)

    LOOKAHEAD = 3

    def _outer(J, carry):
        for j in range(NSLOT):
            g = J * NSLOT + j
            slot = j
            pslot = (j - LOOKAHEAD) % NSLOT

            @pl.when(g >= NSLOT)
            def _drain_scatter():
                pltpu.make_async_copy(h_hbm.at[pl.ds(0, MC)],
                                      rvs[slot], sem_s.at[slot]).wait()

            _issue(g, slot)

            @pl.when(g >= LOOKAHEAD)
            def _do_finish():
                _finish(g - LOOKAHEAD, pslot)
        return carry

    lax.fori_loop(0, MJ, _outer, 0)
    for t in range(LOOKAHEAD, 0, -1):
        _finish(MNCH - t, (MNCH - t) % NSLOT)
    for j in range(NSLOT):
        pltpu.make_async_copy(h_hbm.at[pl.ds(0, MC)], rvs[j],
                              sem_s.at[j]).wait()

    plsc.subcore_barrier()
    pltpu.sync_copy(accum.at[pl.ds(s * RPT, RPT)],
                    out_hbm.at[c, pl.ds(s * RPT, RPT)])

    @pl.when(s == NS - 1)
    def _dump_tail():
        pltpu.sync_copy(accum.at[pl.ds(NS * RPT, TAIL)],
                        out_hbm.at[c, pl.ds(NS * RPT, TAIL)])


def _msgpass_entry(layer):
    def body(h_hbm, src_hbm, dst_hbm, e_hbm, out_hbm, accum,
             src_flat, dst_flat,
             is0, is1, is2, is3, is4, id0, id1, id2, id3, id4,
             rv0, rv1, rv2, rv3, rv4, ev0, ev1, ev2, ev3, ev4,
             sem_g, sem_e, sem_s):
        _msgpass_body(layer, h_hbm, src_hbm, dst_hbm, e_hbm, out_hbm,
                      accum, src_flat, dst_flat,
                      [is0, is1, is2, is3, is4], [id0, id1, id2, id3, id4],
                      [rv0, rv1, rv2, rv3, rv4], [ev0, ev1, ev2, ev3, ev4],
                      sem_g, sem_e, sem_s)
    return body


@functools.cache
def _msgpass_call(layer):
    return pl.kernel(
        _msgpass_entry(layer),
        out_type=jax.ShapeDtypeStruct((NC, N, D), jnp.float32),
        mesh=_sc_mesh(),
        scratch_types=(
            [pltpu.VMEM_SHARED((N, D), jnp.float32),
             pltpu.VMEM((EPT,), jnp.int32),
             pltpu.VMEM((EPT,), jnp.int32)]
            + [pltpu.VMEM((MC,), jnp.int32) for _ in range(2 * NSLOT)]
            + [pltpu.VMEM((MC, D), jnp.float32) for _ in range(NSLOT)]
            + [pltpu.VMEM((MC // 2, D), jnp.uint32) for _ in range(NSLOT)]
            + [pltpu.SemaphoreType.DMA((NSLOT,)),
               pltpu.SemaphoreType.DMA((NSLOT,)),
               pltpu.SemaphoreType.DMA((NSLOT,))]
        ),
    )


_BE = 2000


def _edge_emb_body(ea, w, b, o):
    ee = jnp.dot(ea[:, :DE], w[0], preferred_element_type=jnp.float32) + b[0]
    eo = jnp.dot(ea[:, DE:], w[0], preferred_element_type=jnp.float32) + b[0]
    ue = jax.lax.bitcast_convert_type(
        ee.astype(jnp.bfloat16), jnp.uint16).astype(jnp.uint32)
    uo = jax.lax.bitcast_convert_type(
        eo.astype(jnp.bfloat16), jnp.uint16).astype(jnp.uint32)
    o[0] = ue | (uo << 16)


def _edge_emb(edge_attr2, We, be):
    return pl.pallas_call(
        _edge_emb_body,
        grid=(L, E // 2 // _BE),
        in_specs=[
            pl.BlockSpec((_BE, 2 * DE), lambda l, e: (e, 0)),
            pl.BlockSpec((1, DE, H), lambda l, e: (l, 0, 0)),
            pl.BlockSpec((1, 1, H), lambda l, e: (l, 0, 0)),
        ],
        out_specs=pl.BlockSpec((1, _BE, H), lambda l, e: (l, e, 0)),
        out_shape=jax.ShapeDtypeStruct((L, E // 2, H), jnp.uint32),
    )(edge_attr2, We, be)


_BN = 1000


def _layer_body(sp, cp, h, w1, b1, g1, c1, w2, b2, o):
    cnt = cp[0, :, 0:1] + cp[1, :, 0:1]
    ssum = sp[0] + sp[1] + EPS * cnt
    agg = ssum / jnp.maximum(cnt, 1.0)
    out = agg + h[...]
    t = jnp.dot(out, w1[...], preferred_element_type=jnp.float32) + b1[...]
    t = t * (g1[...] * BN_SCALE) + c1[...]
    t = jnp.maximum(t, 0.0)
    hn = jnp.dot(t, w2[...], preferred_element_type=jnp.float32) + b2[...]
    o[...] = jnp.maximum(hn, 0.0)


def _layer_update(sparts, cparts, h, w1, b1, g1, c1, w2, b2):
    return pl.pallas_call(
        _layer_body,
        grid=(N // _BN,),
        in_specs=[
            pl.BlockSpec((NC, _BN, D), lambda b: (0, b, 0)),
            pl.BlockSpec((NC, _BN, D), lambda b: (0, b, 0)),
            pl.BlockSpec((_BN, D), lambda b: (b, 0)),
            pl.BlockSpec((D, 2 * H), lambda b: (0, 0)),
            pl.BlockSpec((1, 2 * H), lambda b: (0, 0)),
            pl.BlockSpec((1, 2 * H), lambda b: (0, 0)),
            pl.BlockSpec((1, 2 * H), lambda b: (0, 0)),
            pl.BlockSpec((2 * H, H), lambda b: (0, 0)),
            pl.BlockSpec((1, H), lambda b: (0, 0)),
        ],
        out_specs=pl.BlockSpec((_BN, D), lambda b: (b, 0)),
        out_shape=jax.ShapeDtypeStruct((N, D), jnp.float32),
    )(sparts, cparts, h, w1, b1, g1, c1, w2, b2)


def _head_body(h, b, gf, wh, bhh, o):
    hv = h[...]
    bi = b[...]
    rows = []
    for g in range(G):
        m = (bi == g).astype(jnp.float32)
        cnt = jnp.sum(m, axis=0, keepdims=True)
        sm = jnp.sum(hv * m, axis=0, keepdims=True)
        rows.append(jnp.where(cnt > 0.0, sm / jnp.maximum(cnt, 1.0), 0.0))
    pooled = jnp.concatenate(rows, axis=0)
    z = jnp.concatenate([pooled, gf[...]], axis=1)
    logits = jnp.dot(z, wh[...], preferred_element_type=jnp.float32) + bhh[...]
    mx = jnp.max(logits, axis=1, keepdims=True)
    ex = jnp.exp(logits - mx)
    o[...] = ex / jnp.sum(ex, axis=1, keepdims=True)


def _head(h, batch2d, gf, Wh, bh2d):
    return pl.pallas_call(
        _head_body,
        out_shape=jax.ShapeDtypeStruct((G, OUT), jnp.float32),
    )(h, batch2d, gf, Wh, bh2d)


def kernel(x, edge_attr, graph_features, We, be, Wm1, bm1, bnw, bnb,
           Wm2, bm2, Wh, bh, edge_index, batch, num_graphs):
    src = edge_index[0]
    dst = edge_index[1]
    e_all = _edge_emb(edge_attr.reshape(E // 2, 2 * DE), We,
                      be.reshape(L, 1, H))
    cparts = _cnt_kernel()(dst)
    h = x
    for i in range(L):
        sparts = _msgpass_call(i)(h, src, dst, e_all)
        h = _layer_update(
            sparts, cparts, h,
            Wm1[i], bm1[i].reshape(1, -1), bnw[i].reshape(1, -1),
            bnb[i].reshape(1, -1), Wm2[i], bm2[i].reshape(1, -1))
    return _head(h, batch.reshape(-1, 1), graph_features, Wh,
                 bh.reshape(1, -1))
